# baseline (speedup 1.0000x reference)
import functools

import jax
import jax.numpy as jnp
from jax import lax
from jax.experimental import pallas as pl
from jax.experimental.pallas import tpu as pltpu
from jax.experimental.pallas import tpu_sc as plsc

N = 10000
E = 320000
D = 128
H = D // 2

NC = 2
NS = 16
NSPLIT = 2
E2 = E // NSPLIT
EPT = E2 // NS
C = 80
NCHUNK = EPT // C
NROW = N // C


def _kqv_body(x_ref, wk_ref, wq_ref, wv_ref, bk_ref, bq_ref, bv_ref,
              k_out, qv_out):
    xb = x_ref[...]
    k = jnp.dot(xb, wk_ref[...], preferred_element_type=jnp.float32)
    q = jnp.dot(xb, wq_ref[...], preferred_element_type=jnp.float32)
    v = jnp.dot(xb, wv_ref[...], preferred_element_type=jnp.float32)
    ek = jnp.exp(-(k + bk_ref[...]))
    eq = jnp.exp(-(q + bq_ref[...]))
    v = v + bv_ref[...]
    k_out[0] = ek[:, :H]
    k_out[1] = ek[:, H:]
    qv_out[0, :, :H] = eq[:, :H]
    qv_out[0, :, H:] = v[:, :H]
    qv_out[1, :, :H] = eq[:, H:]
    qv_out[1, :, H:] = v[:, H:]


def _edge_body(ea_ref, we_ref, e_out):
    eb = jnp.exp(-jnp.dot(ea_ref[...], we_ref[...],
                          preferred_element_type=jnp.float32))
    e_out[0] = eb[:, :H]
    e_out[1] = eb[:, H:]


def _final_body(part0_ref, part1_ref, x_ref, ws_ref, b_ref, g_ref, beta_ref,
                y_out):
    xb = x_ref[...]
    agg = jnp.concatenate([part0_ref[0] + part1_ref[0],
                           part0_ref[1] + part1_ref[1]], axis=1)
    out = (agg
           + jnp.dot(xb, ws_ref[...], preferred_element_type=jnp.float32)
           + b_ref[...])
    mean = jnp.mean(out, axis=0, keepdims=True)
    cent = out - mean
    var = jnp.mean(cent * cent, axis=0, keepdims=True)
    h = cent * lax.rsqrt(var + 1e-5) * g_ref[...] + beta_ref[...]
    y_out[...] = xb + jnp.maximum(h, 0.0)


def _sc_edge_body(k_hbm, qv_hbm, e_hbm, src_hbm, dst_hbm, out_hbm,
                  srcc, dstc, dstg, kdv0, kdv1, qvv0, qvv1, ein0, ein1,
                  msg0, msg1, agg, si0, si1, sg0, sg1, ss0, ss1):
    cid = lax.axis_index("c")
    sid = lax.axis_index("s")
    cbase = cid * N
    kdv = (kdv0, kdv1)
    qvv = (qvv0, qvv1)
    ein = (ein0, ein1)
    msg = (msg0, msg1)
    si = (si0, si1)
    sg = (sg0, sg1)
    ss = (ss0, ss1)

    def _start_idx(c, u):
        pltpu.async_copy(src_hbm.at[sid, c], srcc.at[u % 2], si[u % 2])
        pltpu.async_copy(dst_hbm.at[sid, c], dstc.at[u % 2], si[u % 2])

    def _wait_idx(u):
        pltpu.make_async_copy(src_hbm.at[0, 0], srcc.at[u % 2],
                              si[u % 2]).wait()
        pltpu.make_async_copy(dst_hbm.at[0, 0], dstc.at[u % 2],
                              si[u % 2]).wait()

    def _adjust(u):
        b = u % 2
        for t in range(C // 16):
            sl = pl.ds(t * 16, 16)
            srcc[b, sl] = srcc[b, sl] + cbase
            dstg[b, sl] = dstc[b, sl] + cbase

    def _start_g(c, u):
        b = u % 2
        pltpu.async_copy(k_hbm.at[dstg.at[b]], kdv[b], sg[b])
        pltpu.async_copy(qv_hbm.at[srcc.at[b]], qvv[b], sg[b])
        pltpu.async_copy(e_hbm.at[cid, pl.ds(sid * EPT + c * C, C)],
                         ein[b], sg[b])

    def _wait_g(u):
        b = u % 2
        pltpu.make_async_copy(k_hbm.at[dstg.at[b]], kdv[b], sg[b]).wait()
        pltpu.make_async_copy(qv_hbm.at[srcc.at[b]], qvv[b], sg[b]).wait()
        pltpu.make_async_copy(e_hbm.at[0, pl.ds(0, C)], ein[b], sg[b]).wait()

    def _compute(u):
        kd, ei, qv, ms = kdv[u % 2], ein[u % 2], qvv[u % 2], msg[u % 2]

        @plsc.parallel_loop(0, C, step=1, unroll=2)
        def _row(i):
            for j in range(H // 16):
                sl = pl.ds(j * 16, 16)
                p = kd[i, sl] * ei[i, sl] * qv[i, sl]
                vv = qv[i, pl.ds(H + j * 16, 16)]
                ms[i, sl] = vv / (1.0 + p)

    def _zrow(i, _):
        for j in range(H // 16):
            msg0[i, pl.ds(j * 16, 16)] = jnp.zeros((16,), jnp.float32)
        return 0
    lax.fori_loop(0, C, _zrow, 0)
    for z in range((NROW + NS - 1) // NS):
        idx = sid + z * NS

        @pl.when(idx < NROW)
        def _():
            pltpu.sync_copy(msg0, agg.at[pl.ds(idx * C, C)])
    plsc.subcore_barrier()

    def _body(c, u, fetch, advance):
        _wait_g(u)
        if advance:
            _wait_idx(u + 1)
            _adjust(u + 1)
            _start_g(c + 1, u + 1)
        _compute(u)
        pltpu.sync_copy(msg[u % 2], agg.at[dstc.at[u % 2]], add=True)
        if fetch:
            _start_idx(c + 2, u)

    _start_idx(0, 0)
    _wait_idx(0)
    _adjust(0)
    _start_g(0, 0)
    _start_idx(1, 1)

    def _pair(g, _):
        for u in range(2):
            _body(2 * g + u, u, fetch=True, advance=True)
        return 0

    lax.fori_loop(0, (NCHUNK - 3) // 2, _pair, 0)
    _body(NCHUNK - 3, 0, fetch=True, advance=True)
    _body(NCHUNK - 2, 1, fetch=False, advance=True)
    _body(NCHUNK - 1, 0, fetch=False, advance=False)
    plsc.subcore_barrier()

    for z in range((NROW + NS - 1) // NS):
        idx = sid + z * NS

        @pl.when(idx < NROW)
        def _():
            pltpu.sync_copy(agg.at[pl.ds(idx * C, C)], msg0)
            pltpu.sync_copy(msg0, out_hbm.at[cid, pl.ds(idx * C, C)])


@functools.cache
def _sc_edge():
    return pl.kernel(
        _sc_edge_body,
        out_type=jax.ShapeDtypeStruct((NC, N, H), jnp.float32),
        mesh=plsc.VectorSubcoreMesh(core_axis_name="c", subcore_axis_name="s",
                                    num_cores=NC, num_subcores=NS),
        compiler_params=pltpu.CompilerParams(use_tc_tiling_on_sc=False),
        scratch_types=[
            pltpu.VMEM((2, C), jnp.int32),
            pltpu.VMEM((2, C), jnp.int32),
            pltpu.VMEM((2, C), jnp.int32),
            pltpu.VMEM((C, H), jnp.float32),
            pltpu.VMEM((C, H), jnp.float32),
            pltpu.VMEM((C, D), jnp.float32),
            pltpu.VMEM((C, D), jnp.float32),
            pltpu.VMEM((C, H), jnp.float32),
            pltpu.VMEM((C, H), jnp.float32),
            pltpu.VMEM((C, H), jnp.float32),
            pltpu.VMEM((C, H), jnp.float32),
            pltpu.VMEM_SHARED((N, H), jnp.float32),
            pltpu.SemaphoreType.DMA,
            pltpu.SemaphoreType.DMA,
            pltpu.SemaphoreType.DMA,
            pltpu.SemaphoreType.DMA,
            pltpu.SemaphoreType.DMA,
            pltpu.SemaphoreType.DMA,
        ],
    )


def kernel(x, edge_index, edge_attr, W_key, b_key, W_query, b_query,
           W_value, b_value, W_edge, W_skip, bias, gamma, beta):
    src = edge_index[0].reshape(NSPLIT, NS, NCHUNK, C)
    dst = edge_index[1].reshape(NSPLIT, NS, NCHUNK, C)
    bk = b_key.reshape(1, D)
    bq = b_query.reshape(1, D)
    bv = b_value.reshape(1, D)

    RB = 1000
    kt, qvt = pl.pallas_call(
        _kqv_body,
        grid=(N // RB,),
        in_specs=[
            pl.BlockSpec((RB, D), lambda i: (i, 0)),
            pl.BlockSpec((D, D), lambda i: (0, 0)),
            pl.BlockSpec((D, D), lambda i: (0, 0)),
            pl.BlockSpec((D, D), lambda i: (0, 0)),
            pl.BlockSpec((1, D), lambda i: (0, 0)),
            pl.BlockSpec((1, D), lambda i: (0, 0)),
            pl.BlockSpec((1, D), lambda i: (0, 0)),
        ],
        out_specs=[
            pl.BlockSpec((NC, RB, H), lambda i: (0, i, 0)),
            pl.BlockSpec((NC, RB, D), lambda i: (0, i, 0)),
        ],
        out_shape=[
            jax.ShapeDtypeStruct((NC, N, H), jnp.float32),
            jax.ShapeDtypeStruct((NC, N, D), jnp.float32),
        ],
    )(x, W_key, W_query, W_value, bk, bq, bv)

    EB = 6400
    kt2 = kt.reshape(NC * N, H)
    qvt2 = qvt.reshape(NC * N, D)
    parts = []
    for h in range(NSPLIT):
        e_h = pl.pallas_call(
            _edge_body,
            grid=(E2 // EB,),
            in_specs=[
                pl.BlockSpec((EB, D), lambda i: (i, 0)),
                pl.BlockSpec((D, D), lambda i: (0, 0)),
            ],
            out_specs=pl.BlockSpec((NC, EB, H), lambda i: (0, i, 0)),
            out_shape=jax.ShapeDtypeStruct((NC, E2, H), jnp.float32),
        )(lax.slice_in_dim(edge_attr, h * E2, (h + 1) * E2, axis=0), W_edge)
        parts.append(_sc_edge()(kt2, qvt2, e_h, src[h], dst[h]))

    y = pl.pallas_call(
        _final_body,
        in_specs=[
            pl.BlockSpec((NC, N, H), lambda: (0, 0, 0)),
            pl.BlockSpec((NC, N, H), lambda: (0, 0, 0)),
            pl.BlockSpec((N, D), lambda: (0, 0)),
            pl.BlockSpec((D, D), lambda: (0, 0)),
            pl.BlockSpec((1, D), lambda: (0, 0)),
            pl.BlockSpec((1, D), lambda: (0, 0)),
            pl.BlockSpec((1, D), lambda: (0, 0)),
        ],
        out_specs=pl.BlockSpec((N, D), lambda: (0, 0)),
        out_shape=jax.ShapeDtypeStruct((N, D), jnp.float32),
    )(parts[0], parts[1], x, W_skip, bias.reshape(1, D), gamma.reshape(1, D),
      beta.reshape(1, D))
    return y


# speedup vs baseline: 1.2006x; 1.2006x over previous
import functools

import jax
import jax.numpy as jnp
import numpy as np
from jax import lax
from jax.experimental import pallas as pl
from jax.experimental.pallas import tpu as pltpu
from jax.experimental.pallas import tpu_sc as plsc

N = 10000
E = 320000
D = 128
H = D // 2
HW = H // 2

NC = 2
NS = 16
NSPLIT = 2
E2 = E // NSPLIT
EPT = E2 // NS
C = 80
NCHUNK = EPT // C
NROW = N // C


def _perm_matrix():
    pm = np.zeros((H, H), np.float32)
    for g in range(H // 32):
        for h in range(2):
            for t in range(16):
                pm[32 * g + 2 * t + h, 32 * g + 16 * h + t] = 1.0
    return pm


def _rn_bf16_bits(x):
    b = lax.bitcast_convert_type(x, jnp.int32)
    return b + 0x7FFF + lax.bitwise_and(lax.shift_right_logical(b, 16), 1)


def _pack_pairs(xp):
    himask = jnp.int32(-65536)
    words = []
    for g in range(H // 32):
        ev = _rn_bf16_bits(xp[:, 32 * g:32 * g + 16])
        od = _rn_bf16_bits(xp[:, 32 * g + 16:32 * g + 32])
        words.append(lax.bitwise_or(lax.shift_right_logical(ev, 16),
                                    lax.bitwise_and(od, himask)))
    return jnp.concatenate(words, axis=1)


def _kqv_body(x_ref, wk_ref, wq_ref, wv_ref, bk_ref, bq_ref, bv_ref, pm_ref,
              ek_out, eq_out, v_out):
    xb = x_ref[...]
    k = jnp.dot(xb, wk_ref[...], preferred_element_type=jnp.float32)
    q = jnp.dot(xb, wq_ref[...], preferred_element_type=jnp.float32)
    v = jnp.dot(xb, wv_ref[...], preferred_element_type=jnp.float32)
    ek = jnp.exp(-(k + bk_ref[...]))
    eq = jnp.exp(-(q + bq_ref[...]))
    v = v + bv_ref[...]
    pm = pm_ref[...]
    for c in range(NC):
        half = slice(c * H, (c + 1) * H)
        ek_out[c] = _pack_pairs(jnp.dot(ek[:, half], pm,
                                        preferred_element_type=jnp.float32))
        eq_out[c] = _pack_pairs(jnp.dot(eq[:, half], pm,
                                        preferred_element_type=jnp.float32))
        v_out[c] = jnp.dot(v[:, half], pm,
                           preferred_element_type=jnp.float32)


def _edge_body(ea_ref, we_ref, pm_ref, e_out):
    eb = jnp.exp(-jnp.dot(ea_ref[...], we_ref[...],
                          preferred_element_type=jnp.float32))
    pm = pm_ref[...]
    for c in range(NC):
        e_out[c] = _pack_pairs(jnp.dot(eb[:, c * H:(c + 1) * H], pm,
                                       preferred_element_type=jnp.float32))


def _final_body(part0_ref, part1_ref, x_ref, ws_ref, b_ref, g_ref, beta_ref,
                pmt_ref, y_out):
    xb = x_ref[...]
    pmt = pmt_ref[...]
    agg = jnp.concatenate(
        [jnp.dot(part0_ref[c] + part1_ref[c], pmt,
                 preferred_element_type=jnp.float32) for c in range(NC)],
        axis=1)
    out = (agg
           + jnp.dot(xb, ws_ref[...], preferred_element_type=jnp.float32)
           + b_ref[...])
    mean = jnp.mean(out, axis=0, keepdims=True)
    cent = out - mean
    var = jnp.mean(cent * cent, axis=0, keepdims=True)
    h = cent * lax.rsqrt(var + 1e-5) * g_ref[...] + beta_ref[...]
    y_out[...] = xb + jnp.maximum(h, 0.0)


def _sc_edge_body(ek_hbm, eq_hbm, v_hbm, e_hbm, src_hbm, dst_hbm, out_hbm,
                  srcc, dstc, dstg, ekv0, ekv1, eqv0, eqv1, vv0, vv1,
                  ein0, ein1, msg0, msg1, agg, si0, si1, sg0, sg1):
    cid = lax.axis_index("c")
    sid = lax.axis_index("s")
    cbase = cid * N
    ekv = (ekv0, ekv1)
    eqv = (eqv0, eqv1)
    vv = (vv0, vv1)
    ein = (ein0, ein1)
    msg = (msg0, msg1)
    si = (si0, si1)
    sg = (sg0, sg1)

    def _start_idx(c, u):
        pltpu.async_copy(src_hbm.at[sid, c], srcc.at[u % 2], si[u % 2])
        pltpu.async_copy(dst_hbm.at[sid, c], dstc.at[u % 2], si[u % 2])

    def _wait_idx(u):
        pltpu.make_async_copy(src_hbm.at[0, 0], srcc.at[u % 2],
                              si[u % 2]).wait()
        pltpu.make_async_copy(dst_hbm.at[0, 0], dstc.at[u % 2],
                              si[u % 2]).wait()

    def _adjust(u):
        b = u % 2
        for t in range(C // 16):
            sl = pl.ds(t * 16, 16)
            srcc[b, sl] = srcc[b, sl] + cbase
            dstg[b, sl] = dstc[b, sl] + cbase

    def _start_g(c, u):
        b = u % 2
        pltpu.async_copy(ek_hbm.at[dstg.at[b]], ekv[b], sg[b])
        pltpu.async_copy(eq_hbm.at[srcc.at[b]], eqv[b], sg[b])
        pltpu.async_copy(v_hbm.at[srcc.at[b]], vv[b], sg[b])
        pltpu.async_copy(e_hbm.at[cid, pl.ds(sid * EPT + c * C, C)],
                         ein[b], sg[b])

    def _wait_g(u):
        b = u % 2
        pltpu.make_async_copy(ek_hbm.at[dstg.at[b]], ekv[b], sg[b]).wait()
        pltpu.make_async_copy(eq_hbm.at[srcc.at[b]], eqv[b], sg[b]).wait()
        pltpu.make_async_copy(v_hbm.at[srcc.at[b]], vv[b], sg[b]).wait()
        pltpu.make_async_copy(e_hbm.at[0, pl.ds(0, C)], ein[b], sg[b]).wait()

    himask = jnp.int32(-65536)

    def _unpk(w):
        lo = lax.bitcast_convert_type(lax.shift_left(w, 16), jnp.float32)
        hi = lax.bitcast_convert_type(lax.bitwise_and(w, himask), jnp.float32)
        return lo, hi

    def _compute(u):
        kd, qe, ei, vb, ms = (ekv[u % 2], eqv[u % 2], ein[u % 2],
                              vv[u % 2], msg[u % 2])

        @plsc.parallel_loop(0, C, step=1, unroll=2)
        def _row(i):
            for g in range(H // 32):
                sl = pl.ds(g * 16, 16)
                ka, kb = _unpk(kd[i, sl])
                qa, qb = _unpk(qe[i, sl])
                ea, eb = _unpk(ei[i, sl])
                sa = pl.ds(g * 32, 16)
                sb = pl.ds(g * 32 + 16, 16)
                ms[i, sa] = vb[i, sa] / (1.0 + ka * ea * qa)
                ms[i, sb] = vb[i, sb] / (1.0 + kb * eb * qb)

    def _zrow(i, _):
        for j in range(H // 16):
            msg0[i, pl.ds(j * 16, 16)] = jnp.zeros((16,), jnp.float32)
        return 0
    lax.fori_loop(0, C, _zrow, 0)
    for z in range((NROW + NS - 1) // NS):
        idx = sid + z * NS

        @pl.when(idx < NROW)
        def _():
            pltpu.sync_copy(msg0, agg.at[pl.ds(idx * C, C)])
    plsc.subcore_barrier()

    def _body(c, u, fetch, advance):
        _wait_g(u)
        if advance:
            _wait_idx(u + 1)
            _adjust(u + 1)
            _start_g(c + 1, u + 1)
        _compute(u)
        pltpu.sync_copy(msg[u % 2], agg.at[dstc.at[u % 2]], add=True)
        if fetch:
            _start_idx(c + 2, u)

    _start_idx(0, 0)
    _wait_idx(0)
    _adjust(0)
    _start_g(0, 0)
    _start_idx(1, 1)

    def _pair(g, _):
        for u in range(2):
            _body(2 * g + u, u, fetch=True, advance=True)
        return 0

    lax.fori_loop(0, (NCHUNK - 3) // 2, _pair, 0)
    _body(NCHUNK - 3, 0, fetch=True, advance=True)
    _body(NCHUNK - 2, 1, fetch=False, advance=True)
    _body(NCHUNK - 1, 0, fetch=False, advance=False)
    plsc.subcore_barrier()

    for z in range((NROW + NS - 1) // NS):
        idx = sid + z * NS

        @pl.when(idx < NROW)
        def _():
            pltpu.sync_copy(agg.at[pl.ds(idx * C, C)], msg0)
            pltpu.sync_copy(msg0, out_hbm.at[cid, pl.ds(idx * C, C)])


@functools.cache
def _sc_edge():
    return pl.kernel(
        _sc_edge_body,
        out_type=jax.ShapeDtypeStruct((NC, N, H), jnp.float32),
        mesh=plsc.VectorSubcoreMesh(core_axis_name="c", subcore_axis_name="s",
                                    num_cores=NC, num_subcores=NS),
        compiler_params=pltpu.CompilerParams(use_tc_tiling_on_sc=False),
        scratch_types=[
            pltpu.VMEM((2, C), jnp.int32),
            pltpu.VMEM((2, C), jnp.int32),
            pltpu.VMEM((2, C), jnp.int32),
            pltpu.VMEM((C, HW), jnp.int32),
            pltpu.VMEM((C, HW), jnp.int32),
            pltpu.VMEM((C, HW), jnp.int32),
            pltpu.VMEM((C, HW), jnp.int32),
            pltpu.VMEM((C, H), jnp.float32),
            pltpu.VMEM((C, H), jnp.float32),
            pltpu.VMEM((C, HW), jnp.int32),
            pltpu.VMEM((C, HW), jnp.int32),
            pltpu.VMEM((C, H), jnp.float32),
            pltpu.VMEM((C, H), jnp.float32),
            pltpu.VMEM_SHARED((N, H), jnp.float32),
            pltpu.SemaphoreType.DMA,
            pltpu.SemaphoreType.DMA,
            pltpu.SemaphoreType.DMA,
            pltpu.SemaphoreType.DMA,
        ],
    )


def kernel(x, edge_index, edge_attr, W_key, b_key, W_query, b_query,
           W_value, b_value, W_edge, W_skip, bias, gamma, beta):
    src = edge_index[0].reshape(NSPLIT, NS, NCHUNK, C)
    dst = edge_index[1].reshape(NSPLIT, NS, NCHUNK, C)
    bk = b_key.reshape(1, D)
    bq = b_query.reshape(1, D)
    bv = b_value.reshape(1, D)
    pm = jnp.asarray(_perm_matrix())
    pmt = jnp.asarray(_perm_matrix().T)

    RB = 1000
    ekt, eqt, vt = pl.pallas_call(
        _kqv_body,
        grid=(N // RB,),
        in_specs=[
            pl.BlockSpec((RB, D), lambda i: (i, 0)),
            pl.BlockSpec((D, D), lambda i: (0, 0)),
            pl.BlockSpec((D, D), lambda i: (0, 0)),
            pl.BlockSpec((D, D), lambda i: (0, 0)),
            pl.BlockSpec((1, D), lambda i: (0, 0)),
            pl.BlockSpec((1, D), lambda i: (0, 0)),
            pl.BlockSpec((1, D), lambda i: (0, 0)),
            pl.BlockSpec((H, H), lambda i: (0, 0)),
        ],
        out_specs=[
            pl.BlockSpec((NC, RB, HW), lambda i: (0, i, 0)),
            pl.BlockSpec((NC, RB, HW), lambda i: (0, i, 0)),
            pl.BlockSpec((NC, RB, H), lambda i: (0, i, 0)),
        ],
        out_shape=[
            jax.ShapeDtypeStruct((NC, N, HW), jnp.int32),
            jax.ShapeDtypeStruct((NC, N, HW), jnp.int32),
            jax.ShapeDtypeStruct((NC, N, H), jnp.float32),
        ],
    )(x, W_key, W_query, W_value, bk, bq, bv, pm)

    EB = 6400
    ekt2 = ekt.reshape(NC * N, HW)
    eqt2 = eqt.reshape(NC * N, HW)
    vt2 = vt.reshape(NC * N, H)
    parts = []
    for h in range(NSPLIT):
        e_h = pl.pallas_call(
            _edge_body,
            grid=(E2 // EB,),
            in_specs=[
                pl.BlockSpec((EB, D),
                             lambda i, hh=h: (i + hh * (E2 // EB), 0)),
                pl.BlockSpec((D, D), lambda i: (0, 0)),
                pl.BlockSpec((H, H), lambda i: (0, 0)),
            ],
            out_specs=pl.BlockSpec((NC, EB, HW), lambda i: (0, i, 0)),
            out_shape=jax.ShapeDtypeStruct((NC, E2, HW), jnp.int32),
        )(edge_attr, W_edge, pm)
        parts.append(_sc_edge()(ekt2, eqt2, vt2, e_h, src[h], dst[h]))

    y = pl.pallas_call(
        _final_body,
        in_specs=[
            pl.BlockSpec((NC, N, H), lambda: (0, 0, 0)),
            pl.BlockSpec((NC, N, H), lambda: (0, 0, 0)),
            pl.BlockSpec((N, D), lambda: (0, 0)),
            pl.BlockSpec((D, D), lambda: (0, 0)),
            pl.BlockSpec((1, D), lambda: (0, 0)),
            pl.BlockSpec((1, D), lambda: (0, 0)),
            pl.BlockSpec((1, D), lambda: (0, 0)),
            pl.BlockSpec((H, H), lambda: (0, 0)),
        ],
        out_specs=pl.BlockSpec((N, D), lambda: (0, 0)),
        out_shape=jax.ShapeDtypeStruct((N, D), jnp.float32),
    )(parts[0], parts[1], x, W_skip, bias.reshape(1, D), gamma.reshape(1, D),
      beta.reshape(1, D), pmt)
    return y
